# Initial kernel scaffold; baseline (speedup 1.0000x reference)
import functools

import jax
import jax.numpy as jnp
from jax import lax
from jax.experimental import pallas as pl
from jax.experimental.pallas import tpu as pltpu
from jax.experimental.pallas import tpu_sc as plsc

F32 = jnp.float32
I32 = jnp.int32

_NC = 2
_NS = 16
_NW = _NC * _NS
_K = 128

_HIGH = lax.Precision.HIGHEST


def _silu(t):
    return t * jax.nn.sigmoid(t)


def _full16(v):
    return jnp.full((16,), v, dtype=I32)


def _make_gather(E, N):
    n_chunks = E // _K
    base_cnt = n_chunks // _NW
    n_extra = n_chunks - base_cnt * _NW
    mesh = plsc.VectorSubcoreMesh(core_axis_name="c", subcore_axis_name="s")

    @functools.partial(
        pl.kernel,
        mesh=mesh,
        out_type=(
            jax.ShapeDtypeStruct((E, 64), F32),
            jax.ShapeDtypeStruct((E, 64), F32),
            jax.ShapeDtypeStruct((E, 8), F32),
        ),
        scratch_types=[
            pltpu.VMEM((N, 4), F32),
            pltpu.VMEM((_K,), I32),
            pltpu.VMEM((_K,), I32),
            pltpu.VMEM((_K, 64), F32),
            pltpu.VMEM((_K, 64), F32),
            pltpu.VMEM((_K, 8), F32),
            pltpu.SemaphoreType.DMA,
            pltpu.SemaphoreType.DMA,
        ],
    )
    def gather_k(pa_hbm, pb_hbm, x4_hbm, row_hbm, col_hbm,
                 ga_hbm, gb_hbm, rc_hbm,
                 xtab, rowi, coli, bufa, bufb, rcbuf, sema, semb):
        cid = lax.axis_index("c")
        sid = lax.axis_index("s")
        wid = sid * _NC + cid
        cnt = base_cnt + jnp.where(wid < n_extra, 1, 0)
        pltpu.sync_copy(x4_hbm, xtab)

        @pl.loop(0, base_cnt + 1)
        def _(i):
            @pl.when(i < cnt)
            def _():
                base = (wid + _NW * i) * _K
                pltpu.sync_copy(row_hbm.at[pl.ds(base, _K)], rowi)
                pltpu.sync_copy(col_hbm.at[pl.ds(base, _K)], coli)
                cpa = pltpu.async_copy(pa_hbm.at[rowi], bufa, sema)
                cpb = pltpu.async_copy(pb_hbm.at[coli], bufb, semb)

                @pl.loop(0, _K, step=16)
                def _(g):
                    rv = rowi[pl.ds(g, 16)]
                    cv = coli[pl.ds(g, 16)]
                    d0 = (plsc.load_gather(xtab, [rv, _full16(0)])
                          - plsc.load_gather(xtab, [cv, _full16(0)]))
                    d1 = (plsc.load_gather(xtab, [rv, _full16(1)])
                          - plsc.load_gather(xtab, [cv, _full16(1)]))
                    d2 = (plsc.load_gather(xtab, [rv, _full16(2)])
                          - plsc.load_gather(xtab, [cv, _full16(2)]))
                    rad = d0 * d0 + d1 * d1 + d2 * d2
                    rows = lax.iota(I32, 16) + g
                    plsc.store_scatter(rcbuf, [rows, _full16(0)], rad)
                    plsc.store_scatter(rcbuf, [rows, _full16(1)], d0)
                    plsc.store_scatter(rcbuf, [rows, _full16(2)], d1)
                    plsc.store_scatter(rcbuf, [rows, _full16(3)], d2)

                cpa.wait()
                cpb.wait()
                pltpu.sync_copy(bufa, ga_hbm.at[pl.ds(base, _K)])
                pltpu.sync_copy(bufb, gb_hbm.at[pl.ds(base, _K)])
                pltpu.sync_copy(rcbuf, rc_hbm.at[pl.ds(base, _K)])

    return gather_k


def _make_scatter(E, N):
    n_chunks = E // _K
    base_cnt = n_chunks // _NW
    n_extra = n_chunks - base_cnt * _NW
    rows_per_tile = N // _NS
    zrows = rows_per_tile // 5
    mesh = plsc.VectorSubcoreMesh(core_axis_name="c", subcore_axis_name="s")

    @functools.partial(
        pl.kernel,
        mesh=mesh,
        out_type=(
            jax.ShapeDtypeStruct((_NC, N, 64), F32),
            jax.ShapeDtypeStruct((_NC, N, 16), F32),
        ),
        scratch_types=[
            pltpu.VMEM_SHARED((N, 64), F32),
            pltpu.VMEM_SHARED((N, 16), F32),
            pltpu.VMEM((1, _K), I32),
            pltpu.VMEM((_K, 64), F32),
            pltpu.VMEM((_K, 16), F32),
            pltpu.VMEM((200, 64), F32),
            pltpu.VMEM((200, 16), F32),
        ],
    )
    def scatter_k(row_hbm, oem_hbm, oet_hbm, pm_hbm, pt_hbm,
                  accm, acct, rowi, ebm, ebt, zbm, zbt):
        cid = lax.axis_index("c")
        sid = lax.axis_index("s")
        wid = sid * _NC + cid
        cnt = base_cnt + jnp.where(wid < n_extra, 1, 0)

        z16 = jnp.zeros((16,), F32)

        @pl.loop(0, 200)
        def _(r):
            @pl.loop(0, 64, step=16)
            def _(c):
                zbm[r, pl.ds(c, 16)] = z16
            zbt[r, pl.ds(0, 16)] = z16

        nz_full = rows_per_tile // 200
        rem = rows_per_tile - nz_full * 200

        @pl.loop(0, nz_full)
        def _(j):
            off = sid * rows_per_tile + j * 200
            pltpu.sync_copy(zbm, accm.at[pl.ds(off, 200)])
            pltpu.sync_copy(zbt, acct.at[pl.ds(off, 200)])

        if rem:
            off = sid * rows_per_tile + nz_full * 200
            pltpu.sync_copy(zbm.at[pl.ds(0, rem)], accm.at[pl.ds(off, rem)])
            pltpu.sync_copy(zbt.at[pl.ds(0, rem)], acct.at[pl.ds(off, rem)])

        plsc.subcore_barrier()

        @pl.loop(0, base_cnt + 1)
        def _(i):
            @pl.when(i < cnt)
            def _():
                base = (wid + _NW * i) * _K
                pltpu.sync_copy(row_hbm.at[pl.ds(base, _K)], rowi.at[0])
                pltpu.sync_copy(oem_hbm.at[pl.ds(base, _K)], ebm)
                pltpu.sync_copy(oet_hbm.at[pl.ds(base, _K)], ebt)
                pltpu.sync_copy(ebm, accm.at[rowi.at[0]], add=True)
                pltpu.sync_copy(ebt, acct.at[rowi.at[0]], add=True)

        plsc.subcore_barrier()

        off = sid * rows_per_tile
        pltpu.sync_copy(accm.at[pl.ds(off, rows_per_tile)],
                        pm_hbm.at[cid, pl.ds(off, rows_per_tile)])
        pltpu.sync_copy(acct.at[pl.ds(off, rows_per_tile)],
                        pt_hbm.at[cid, pl.ds(off, rows_per_tile)])

    return scatter_k


def _init_tc(his, emb_W, emb_b, W1a, W1b):
    N, D = his.shape
    BN = 1000

    def body(his_r, ew_r, eb_r, wa_r, wb_r, h_r, pa_r, pb_r):
        h = jnp.dot(his_r[...], ew_r[...], precision=_HIGH) + eb_r[...]
        h_r[...] = h
        pa_r[...] = jnp.dot(h, wa_r[...], precision=_HIGH)
        pb_r[...] = jnp.dot(h, wb_r[...], precision=_HIGH)

    out = jax.ShapeDtypeStruct((N, 64), F32)
    return pl.pallas_call(
        body,
        grid=(N // BN,),
        in_specs=[
            pl.BlockSpec((BN, D), lambda i: (i, 0)),
            pl.BlockSpec((D, 64), lambda i: (0, 0)),
            pl.BlockSpec((1, 64), lambda i: (0, 0)),
            pl.BlockSpec((64, 64), lambda i: (0, 0)),
            pl.BlockSpec((64, 64), lambda i: (0, 0)),
        ],
        out_specs=[
            pl.BlockSpec((BN, 64), lambda i: (i, 0)),
            pl.BlockSpec((BN, 64), lambda i: (i, 0)),
            pl.BlockSpec((BN, 64), lambda i: (i, 0)),
        ],
        out_shape=[out, out, out],
    )(his, emb_W, emb_b, W1a, W1b)


def _edge_tc(ga, gb, rc, edge_attr, W1d, b1, w1c, W2, b2, cW1, cb1, cW2r):
    E = ga.shape[0]
    BE = 512

    def body(ga_r, gb_r, rc_r, ea_r, w1d_r, b1_r, w1c_r, w2_r, b2_r,
             cw1_r, cb1_r, cw2_r, om_r, ot_r):
        rcv = rc_r[...]
        pre = (ga_r[...] + gb_r[...] + rcv[:, 0:1] * w1c_r[...]
               + jnp.dot(ea_r[...], w1d_r[...], precision=_HIGH) + b1_r[...])
        m1 = _silu(pre)
        m = _silu(jnp.dot(m1, w2_r[...], precision=_HIGH) + b2_r[...])
        t = _silu(jnp.dot(m, cw1_r[...], precision=_HIGH) + cb1_r[...])
        phi = jnp.sum(t * cw2_r[...], axis=1, keepdims=True)
        om_r[...] = m
        trans = rcv[:, 1:4] * phi
        ot_r[...] = jnp.concatenate(
            [trans, jnp.ones((BE, 1), F32), jnp.zeros((BE, 12), F32)], axis=1)

    return pl.pallas_call(
        body,
        grid=(E // BE,),
        in_specs=[
            pl.BlockSpec((BE, 64), lambda i: (i, 0)),
            pl.BlockSpec((BE, 64), lambda i: (i, 0)),
            pl.BlockSpec((BE, 8), lambda i: (i, 0)),
            pl.BlockSpec((BE, 16), lambda i: (i, 0)),
            pl.BlockSpec((16, 64), lambda i: (0, 0)),
            pl.BlockSpec((1, 64), lambda i: (0, 0)),
            pl.BlockSpec((1, 64), lambda i: (0, 0)),
            pl.BlockSpec((64, 64), lambda i: (0, 0)),
            pl.BlockSpec((1, 64), lambda i: (0, 0)),
            pl.BlockSpec((64, 64), lambda i: (0, 0)),
            pl.BlockSpec((1, 64), lambda i: (0, 0)),
            pl.BlockSpec((1, 64), lambda i: (0, 0)),
        ],
        out_specs=[
            pl.BlockSpec((BE, 64), lambda i: (i, 0)),
            pl.BlockSpec((BE, 16), lambda i: (i, 0)),
        ],
        out_shape=[
            jax.ShapeDtypeStruct((E, 64), F32),
            jax.ShapeDtypeStruct((E, 16), F32),
        ],
    )(ga, gb, rc, edge_attr, W1d, b1, w1c, W2, b2, cW1, cb1, cW2r)


def _node_tc(pm, pt, h, x4, v4, vW1, vb1, vW2r, vb2, nW1a, nW1b, nb1,
             nW2, nb2, W1a, W1b):
    N = h.shape[0]
    BN = 1000

    def body(pm_r, pt_r, h_r, x4_r, v4_r, vw1_r, vb1_r, vw2_r, vb2_r,
             nwa_r, nwb_r, nb1_r, nw2_r, nb2_r, wa_r, wb_r,
             hn_r, xn_r, vn_r, pan_r, pbn_r):
        h = h_r[...]
        pmv = pm_r[...]
        ptv = pt_r[...]
        aggm = pmv[0] + pmv[1]
        aggt = ptv[0] + ptv[1]
        cntv = aggt[:, 3:4]
        agg4 = aggt[:, 0:4] / jnp.maximum(cntv, 1.0)
        vs1 = _silu(jnp.dot(h, vw1_r[...], precision=_HIGH) + vb1_r[...])
        vscale = jnp.sum(vs1 * vw2_r[...], axis=1, keepdims=True) + vb2_r[...]
        v4n = vscale * v4_r[...] + agg4
        x4n = x4_r[...] + v4n
        hn1 = _silu(jnp.dot(h, nwa_r[...], precision=_HIGH)
                    + jnp.dot(aggm, nwb_r[...], precision=_HIGH) + nb1_r[...])
        hnn = jnp.dot(hn1, nw2_r[...], precision=_HIGH) + nb2_r[...]
        h2 = 2.0 * h + hnn
        hn_r[...] = h2
        xn_r[...] = x4n
        vn_r[...] = v4n
        pan_r[...] = jnp.dot(h2, wa_r[...], precision=_HIGH)
        pbn_r[...] = jnp.dot(h2, wb_r[...], precision=_HIGH)

    w64 = pl.BlockSpec((64, 64), lambda i: (0, 0))
    b64 = pl.BlockSpec((1, 64), lambda i: (0, 0))
    o64 = jax.ShapeDtypeStruct((N, 64), F32)
    o4 = jax.ShapeDtypeStruct((N, 4), F32)
    return pl.pallas_call(
        body,
        grid=(N // BN,),
        in_specs=[
            pl.BlockSpec((2, BN, 64), lambda i: (0, i, 0)),
            pl.BlockSpec((2, BN, 16), lambda i: (0, i, 0)),
            pl.BlockSpec((BN, 64), lambda i: (i, 0)),
            pl.BlockSpec((BN, 4), lambda i: (i, 0)),
            pl.BlockSpec((BN, 4), lambda i: (i, 0)),
            w64, b64, b64,
            pl.BlockSpec((1, 1), lambda i: (0, 0)),
            w64, w64, b64, w64, b64, w64, w64,
        ],
        out_specs=[
            pl.BlockSpec((BN, 64), lambda i: (i, 0)),
            pl.BlockSpec((BN, 4), lambda i: (i, 0)),
            pl.BlockSpec((BN, 4), lambda i: (i, 0)),
            pl.BlockSpec((BN, 64), lambda i: (i, 0)),
            pl.BlockSpec((BN, 64), lambda i: (i, 0)),
        ],
        out_shape=[o64, o4, o4, o64, o64],
    )(pm, pt, h, x4, v4, vW1, vb1, vW2r, vb2, nW1a, nW1b, nb1, nW2, nb2,
      W1a, W1b)


def kernel(his, x, v, edges, edge_attr, params):
    p = params
    N = his.shape[0]
    E = edge_attr.shape[0]
    row = edges[0]
    col = edges[1]

    W1 = p['e_W1']
    W1a, W1b = W1[0:64], W1[64:128]
    w1c = W1[128:129]
    W1d = W1[129:145]
    nW1 = p['n_W1']
    nW1a, nW1b = nW1[0:64], nW1[64:128]
    emb_b = p['emb_b'].reshape(1, 64)
    e_b1 = p['e_b1'].reshape(1, 64)
    e_b2 = p['e_b2'].reshape(1, 64)
    c_b1 = p['c_b1'].reshape(1, 64)
    cW2r = p['c_W2'].reshape(1, 64)
    v_b1 = p['v_b1'].reshape(1, 64)
    vW2r = p['v_W2'].reshape(1, 64)
    v_b2 = p['v_b2'].reshape(1, 1)
    n_b1 = p['n_b1'].reshape(1, 64)
    n_b2 = p['n_b2'].reshape(1, 64)

    x4 = jnp.pad(x, ((0, 0), (0, 1)))
    v4 = jnp.pad(v, ((0, 0), (0, 1)))

    gather_k = _make_gather(E, N)
    scatter_k = _make_scatter(E, N)

    h, pa, pb = _init_tc(his, p['emb_W'], emb_b, W1a, W1b)

    for _ in range(3):
        ga, gb, rc = gather_k(pa, pb, x4, row, col)
        oem, oet = _edge_tc(ga, gb, rc, edge_attr, W1d, e_b1, w1c,
                            p['e_W2'], e_b2, p['c_W1'], c_b1, cW2r)
        pm, pt = scatter_k(row, oem, oet)
        h, x4, v4, pa, pb = _node_tc(pm, pt, h, x4, v4, p['v_W1'], v_b1,
                                     vW2r, v_b2, nW1a, nW1b, n_b1,
                                     p['n_W2'], n_b2, W1a, W1b)

    return (x4[:, :3], h, v4[:, :3])


# speedup vs baseline: 2.0339x; 2.0339x over previous
import dataclasses
import functools

import jax
import jax.numpy as jnp
from jax import lax
from jax.experimental import pallas as pl
from jax.experimental.pallas import tpu as pltpu
from jax.experimental.pallas import tpu_sc as plsc

F32 = jnp.float32
I32 = jnp.int32

_NC = 2
_NS = 16
_NW = _NC * _NS
_K = 128

_HIGH = lax.Precision.HIGHEST


def _silu(t):
    return t * jax.nn.sigmoid(t)


def _full16(v):
    return jnp.full((16,), v, dtype=I32)


def _sc_params():
    cp = pltpu.CompilerParams()
    fields = pltpu.CompilerParams.__dataclass_fields__
    if "needs_layout_passes" in fields:
        cp = dataclasses.replace(cp, needs_layout_passes=False)
    if "use_tc_tiling_on_sc" in fields:
        cp = dataclasses.replace(cp, use_tc_tiling_on_sc=False)
    return cp


def _make_gather(E, N):
    n_chunks = E // _K
    base_cnt = n_chunks // _NW
    n_extra = n_chunks - base_cnt * _NW
    mesh = plsc.VectorSubcoreMesh(core_axis_name="c", subcore_axis_name="s")

    @functools.partial(
        pl.kernel,
        mesh=mesh,
        out_type=(
            jax.ShapeDtypeStruct((E, 64), F32),
            jax.ShapeDtypeStruct((E, 64), F32),
            jax.ShapeDtypeStruct((E, 8), F32),
        ),
        scratch_types=[
            pltpu.VMEM((N, 4), F32),
            pltpu.VMEM((_K,), I32),
            pltpu.VMEM((_K,), I32),
            pltpu.VMEM((_K, 64), F32),
            pltpu.VMEM((_K, 64), F32),
            pltpu.VMEM((_K, 8), F32),
            pltpu.SemaphoreType.DMA,
            pltpu.SemaphoreType.DMA,
        ],
        compiler_params=_sc_params(),
    )
    def gather_k(pa_hbm, pb_hbm, x4_hbm, row_hbm, col_hbm,
                 ga_hbm, gb_hbm, rc_hbm,
                 xtab, rowi, coli, bufa, bufb, rcbuf, sema, semb):
        cid = lax.axis_index("c")
        sid = lax.axis_index("s")
        wid = sid * _NC + cid
        cnt = base_cnt + jnp.where(wid < n_extra, 1, 0)
        pltpu.sync_copy(x4_hbm, xtab)

        @pl.loop(0, base_cnt + 1)
        def _(i):
            @pl.when(i < cnt)
            def _():
                base = (wid + _NW * i) * _K
                pltpu.sync_copy(row_hbm.at[pl.ds(base, _K)], rowi)
                pltpu.sync_copy(col_hbm.at[pl.ds(base, _K)], coli)
                cpa = pltpu.async_copy(pa_hbm.at[rowi], bufa, sema)
                cpb = pltpu.async_copy(pb_hbm.at[coli], bufb, semb)

                @pl.loop(0, _K, step=16)
                def _(g):
                    rv = rowi[pl.ds(g, 16)]
                    cv = coli[pl.ds(g, 16)]
                    d0 = (plsc.load_gather(xtab, [rv, _full16(0)])
                          - plsc.load_gather(xtab, [cv, _full16(0)]))
                    d1 = (plsc.load_gather(xtab, [rv, _full16(1)])
                          - plsc.load_gather(xtab, [cv, _full16(1)]))
                    d2 = (plsc.load_gather(xtab, [rv, _full16(2)])
                          - plsc.load_gather(xtab, [cv, _full16(2)]))
                    rad = d0 * d0 + d1 * d1 + d2 * d2
                    rows = lax.iota(I32, 16) + g
                    plsc.store_scatter(rcbuf, [rows, _full16(0)], rad)
                    plsc.store_scatter(rcbuf, [rows, _full16(1)], d0)
                    plsc.store_scatter(rcbuf, [rows, _full16(2)], d1)
                    plsc.store_scatter(rcbuf, [rows, _full16(3)], d2)

                cpa.wait()
                cpb.wait()
                pltpu.sync_copy(bufa, ga_hbm.at[pl.ds(base, _K)])
                pltpu.sync_copy(bufb, gb_hbm.at[pl.ds(base, _K)])
                pltpu.sync_copy(rcbuf, rc_hbm.at[pl.ds(base, _K)])

    return gather_k


def _make_scatter(E, N):
    n_chunks = E // _K
    base_cnt = n_chunks // _NW
    n_extra = n_chunks - base_cnt * _NW
    rows_per_tile = N // _NS
    zrows = rows_per_tile // 5
    mesh = plsc.VectorSubcoreMesh(core_axis_name="c", subcore_axis_name="s")

    @functools.partial(
        pl.kernel,
        mesh=mesh,
        out_type=(
            jax.ShapeDtypeStruct((_NC, N, 64), F32),
            jax.ShapeDtypeStruct((_NC, N, 16), F32),
        ),
        scratch_types=[
            pltpu.VMEM_SHARED((N, 64), F32),
            pltpu.VMEM_SHARED((N, 16), F32),
            pltpu.VMEM((1, _K), I32),
            pltpu.VMEM((_K, 64), F32),
            pltpu.VMEM((_K, 16), F32),
            pltpu.VMEM((200, 64), F32),
            pltpu.VMEM((200, 16), F32),
        ],
        compiler_params=_sc_params(),
    )
    def scatter_k(row_hbm, oem_hbm, oet_hbm, pm_hbm, pt_hbm,
                  accm, acct, rowi, ebm, ebt, zbm, zbt):
        cid = lax.axis_index("c")
        sid = lax.axis_index("s")
        wid = sid * _NC + cid
        cnt = base_cnt + jnp.where(wid < n_extra, 1, 0)

        z16 = jnp.zeros((16,), F32)

        @pl.loop(0, 200)
        def _(r):
            @pl.loop(0, 64, step=16)
            def _(c):
                zbm[r, pl.ds(c, 16)] = z16
            zbt[r, pl.ds(0, 16)] = z16

        nz_full = rows_per_tile // 200
        rem = rows_per_tile - nz_full * 200

        @pl.loop(0, nz_full)
        def _(j):
            off = sid * rows_per_tile + j * 200
            pltpu.sync_copy(zbm, accm.at[pl.ds(off, 200)])
            pltpu.sync_copy(zbt, acct.at[pl.ds(off, 200)])

        if rem:
            off = sid * rows_per_tile + nz_full * 200
            pltpu.sync_copy(zbm.at[pl.ds(0, rem)], accm.at[pl.ds(off, rem)])
            pltpu.sync_copy(zbt.at[pl.ds(0, rem)], acct.at[pl.ds(off, rem)])

        plsc.subcore_barrier()

        @pl.loop(0, base_cnt + 1)
        def _(i):
            @pl.when(i < cnt)
            def _():
                base = (wid + _NW * i) * _K
                pltpu.sync_copy(row_hbm.at[pl.ds(base, _K)], rowi.at[0])
                pltpu.sync_copy(oem_hbm.at[pl.ds(base, _K)], ebm)
                pltpu.sync_copy(oet_hbm.at[pl.ds(base, _K)], ebt)
                pltpu.sync_copy(ebm, accm.at[rowi.at[0]], add=True)
                pltpu.sync_copy(ebt, acct.at[rowi.at[0]], add=True)

        plsc.subcore_barrier()

        off = sid * rows_per_tile
        pltpu.sync_copy(accm.at[pl.ds(off, rows_per_tile)],
                        pm_hbm.at[cid, pl.ds(off, rows_per_tile)])
        pltpu.sync_copy(acct.at[pl.ds(off, rows_per_tile)],
                        pt_hbm.at[cid, pl.ds(off, rows_per_tile)])

    return scatter_k


def _init_tc(his, emb_W, emb_b, W1a, W1b):
    N, D = his.shape
    BN = 1000

    def body(his_r, ew_r, eb_r, wa_r, wb_r, h_r, pa_r, pb_r):
        h = jnp.dot(his_r[...], ew_r[...], precision=_HIGH) + eb_r[...]
        h_r[...] = h
        pa_r[...] = jnp.dot(h, wa_r[...], precision=_HIGH)
        pb_r[...] = jnp.dot(h, wb_r[...], precision=_HIGH)

    out = jax.ShapeDtypeStruct((N, 64), F32)
    return pl.pallas_call(
        body,
        grid=(N // BN,),
        in_specs=[
            pl.BlockSpec((BN, D), lambda i: (i, 0)),
            pl.BlockSpec((D, 64), lambda i: (0, 0)),
            pl.BlockSpec((1, 64), lambda i: (0, 0)),
            pl.BlockSpec((64, 64), lambda i: (0, 0)),
            pl.BlockSpec((64, 64), lambda i: (0, 0)),
        ],
        out_specs=[
            pl.BlockSpec((BN, 64), lambda i: (i, 0)),
            pl.BlockSpec((BN, 64), lambda i: (i, 0)),
            pl.BlockSpec((BN, 64), lambda i: (i, 0)),
        ],
        out_shape=[out, out, out],
    )(his, emb_W, emb_b, W1a, W1b)


def _edge_tc(ga, gb, rc, edge_attr, W1d, b1, w1c, W2, b2, cW1, cb1, cW2r):
    E = ga.shape[0]
    BE = 512

    def body(ga_r, gb_r, rc_r, ea_r, w1d_r, b1_r, w1c_r, w2_r, b2_r,
             cw1_r, cb1_r, cw2_r, om_r, ot_r):
        rcv = rc_r[...]
        pre = (ga_r[...] + gb_r[...] + rcv[:, 0:1] * w1c_r[...]
               + jnp.dot(ea_r[...], w1d_r[...], precision=_HIGH) + b1_r[...])
        m1 = _silu(pre)
        m = _silu(jnp.dot(m1, w2_r[...], precision=_HIGH) + b2_r[...])
        t = _silu(jnp.dot(m, cw1_r[...], precision=_HIGH) + cb1_r[...])
        phi = jnp.sum(t * cw2_r[...], axis=1, keepdims=True)
        om_r[...] = m
        trans = rcv[:, 1:4] * phi
        ot_r[...] = jnp.concatenate(
            [trans, jnp.ones((BE, 1), F32), jnp.zeros((BE, 12), F32)], axis=1)

    return pl.pallas_call(
        body,
        grid=(E // BE,),
        in_specs=[
            pl.BlockSpec((BE, 64), lambda i: (i, 0)),
            pl.BlockSpec((BE, 64), lambda i: (i, 0)),
            pl.BlockSpec((BE, 8), lambda i: (i, 0)),
            pl.BlockSpec((BE, 16), lambda i: (i, 0)),
            pl.BlockSpec((16, 64), lambda i: (0, 0)),
            pl.BlockSpec((1, 64), lambda i: (0, 0)),
            pl.BlockSpec((1, 64), lambda i: (0, 0)),
            pl.BlockSpec((64, 64), lambda i: (0, 0)),
            pl.BlockSpec((1, 64), lambda i: (0, 0)),
            pl.BlockSpec((64, 64), lambda i: (0, 0)),
            pl.BlockSpec((1, 64), lambda i: (0, 0)),
            pl.BlockSpec((1, 64), lambda i: (0, 0)),
        ],
        out_specs=[
            pl.BlockSpec((BE, 64), lambda i: (i, 0)),
            pl.BlockSpec((BE, 16), lambda i: (i, 0)),
        ],
        out_shape=[
            jax.ShapeDtypeStruct((E, 64), F32),
            jax.ShapeDtypeStruct((E, 16), F32),
        ],
    )(ga, gb, rc, edge_attr, W1d, b1, w1c, W2, b2, cW1, cb1, cW2r)


def _node_tc(pm, pt, h, x4, v4, vW1, vb1, vW2r, vb2, nW1a, nW1b, nb1,
             nW2, nb2, W1a, W1b):
    N = h.shape[0]
    BN = 1000

    def body(pm_r, pt_r, h_r, x4_r, v4_r, vw1_r, vb1_r, vw2_r, vb2_r,
             nwa_r, nwb_r, nb1_r, nw2_r, nb2_r, wa_r, wb_r,
             hn_r, xn_r, vn_r, pan_r, pbn_r):
        h = h_r[...]
        pmv = pm_r[...]
        ptv = pt_r[...]
        aggm = pmv[0] + pmv[1]
        aggt = ptv[0] + ptv[1]
        cntv = aggt[:, 3:4]
        agg4 = aggt[:, 0:4] / jnp.maximum(cntv, 1.0)
        vs1 = _silu(jnp.dot(h, vw1_r[...], precision=_HIGH) + vb1_r[...])
        vscale = jnp.sum(vs1 * vw2_r[...], axis=1, keepdims=True) + vb2_r[...]
        v4n = vscale * v4_r[...] + agg4
        x4n = x4_r[...] + v4n
        hn1 = _silu(jnp.dot(h, nwa_r[...], precision=_HIGH)
                    + jnp.dot(aggm, nwb_r[...], precision=_HIGH) + nb1_r[...])
        hnn = jnp.dot(hn1, nw2_r[...], precision=_HIGH) + nb2_r[...]
        h2 = 2.0 * h + hnn
        hn_r[...] = h2
        xn_r[...] = x4n
        vn_r[...] = v4n
        pan_r[...] = jnp.dot(h2, wa_r[...], precision=_HIGH)
        pbn_r[...] = jnp.dot(h2, wb_r[...], precision=_HIGH)

    w64 = pl.BlockSpec((64, 64), lambda i: (0, 0))
    b64 = pl.BlockSpec((1, 64), lambda i: (0, 0))
    o64 = jax.ShapeDtypeStruct((N, 64), F32)
    o4 = jax.ShapeDtypeStruct((N, 4), F32)
    return pl.pallas_call(
        body,
        grid=(N // BN,),
        in_specs=[
            pl.BlockSpec((2, BN, 64), lambda i: (0, i, 0)),
            pl.BlockSpec((2, BN, 16), lambda i: (0, i, 0)),
            pl.BlockSpec((BN, 64), lambda i: (i, 0)),
            pl.BlockSpec((BN, 4), lambda i: (i, 0)),
            pl.BlockSpec((BN, 4), lambda i: (i, 0)),
            w64, b64, b64,
            pl.BlockSpec((1, 1), lambda i: (0, 0)),
            w64, w64, b64, w64, b64, w64, w64,
        ],
        out_specs=[
            pl.BlockSpec((BN, 64), lambda i: (i, 0)),
            pl.BlockSpec((BN, 4), lambda i: (i, 0)),
            pl.BlockSpec((BN, 4), lambda i: (i, 0)),
            pl.BlockSpec((BN, 64), lambda i: (i, 0)),
            pl.BlockSpec((BN, 64), lambda i: (i, 0)),
        ],
        out_shape=[o64, o4, o4, o64, o64],
    )(pm, pt, h, x4, v4, vW1, vb1, vW2r, vb2, nW1a, nW1b, nb1, nW2, nb2,
      W1a, W1b)


def kernel(his, x, v, edges, edge_attr, params):
    p = params
    N = his.shape[0]
    E = edge_attr.shape[0]
    row = edges[0]
    col = edges[1]

    W1 = p['e_W1']
    W1a, W1b = W1[0:64], W1[64:128]
    w1c = W1[128:129]
    W1d = W1[129:145]
    nW1 = p['n_W1']
    nW1a, nW1b = nW1[0:64], nW1[64:128]
    emb_b = p['emb_b'].reshape(1, 64)
    e_b1 = p['e_b1'].reshape(1, 64)
    e_b2 = p['e_b2'].reshape(1, 64)
    c_b1 = p['c_b1'].reshape(1, 64)
    cW2r = p['c_W2'].reshape(1, 64)
    v_b1 = p['v_b1'].reshape(1, 64)
    vW2r = p['v_W2'].reshape(1, 64)
    v_b2 = p['v_b2'].reshape(1, 1)
    n_b1 = p['n_b1'].reshape(1, 64)
    n_b2 = p['n_b2'].reshape(1, 64)

    x4 = jnp.pad(x, ((0, 0), (0, 1)))
    v4 = jnp.pad(v, ((0, 0), (0, 1)))

    gather_k = _make_gather(E, N)
    scatter_k = _make_scatter(E, N)

    h, pa, pb = _init_tc(his, p['emb_W'], emb_b, W1a, W1b)

    for _ in range(3):
        ga, gb, rc = gather_k(pa, pb, x4, row, col)
        oem, oet = _edge_tc(ga, gb, rc, edge_attr, W1d, e_b1, w1c,
                            p['e_W2'], e_b2, p['c_W1'], c_b1, cW2r)
        pm, pt = scatter_k(row, oem, oet)
        h, x4, v4, pa, pb = _node_tc(pm, pt, h, x4, v4, p['v_W1'], v_b1,
                                     vW2r, v_b2, nW1a, nW1b, n_b1,
                                     p['n_W2'], n_b2, W1a, W1b)

    return (x4[:, :3], h, v4[:, :3])


# speedup vs baseline: 3.5321x; 1.7366x over previous
import dataclasses
import functools

import jax
import jax.numpy as jnp
from jax import lax
from jax.experimental import pallas as pl
from jax.experimental.pallas import tpu as pltpu
from jax.experimental.pallas import tpu_sc as plsc

F32 = jnp.float32
I32 = jnp.int32

_NC = 2
_NS = 16
_NW = _NC * _NS
_K = 128

_PREC = lax.Precision.HIGHEST


def _silu(t):
    return t * jax.nn.sigmoid(t)


def _full16(v):
    return jnp.full((16,), v, dtype=I32)


def _sc_params(tc_tiling=True):
    cp = pltpu.CompilerParams()
    fields = pltpu.CompilerParams.__dataclass_fields__
    if "needs_layout_passes" in fields:
        cp = dataclasses.replace(cp, needs_layout_passes=False)
    if not tc_tiling and "use_tc_tiling_on_sc" in fields:
        cp = dataclasses.replace(cp, use_tc_tiling_on_sc=False)
    return cp


def _make_gather(E, N):
    n_chunks = E // _K
    base_cnt = n_chunks // _NW
    n_extra = n_chunks - base_cnt * _NW
    mesh = plsc.VectorSubcoreMesh(core_axis_name="c", subcore_axis_name="s")

    @functools.partial(
        pl.kernel,
        mesh=mesh,
        out_type=jax.ShapeDtypeStruct((E, 128), F32),
        scratch_types=[
            pltpu.VMEM((N, 4), F32),
            pltpu.VMEM((_K,), I32),
            pltpu.VMEM((_K,), I32),
            pltpu.VMEM((_K, 128), F32),
            pltpu.VMEM((_K, 128), F32),
            pltpu.SemaphoreType.DMA,
            pltpu.SemaphoreType.DMA,
        ],
        compiler_params=_sc_params(tc_tiling=False),
    )
    def gather_k(pab_hbm, x4_hbm, row_hbm, col_hbm, gpre_hbm,
                 xtab, rowi, coli, bufr, bufc, sema, semb):
        cid = lax.axis_index("c")
        sid = lax.axis_index("s")
        wid = sid * _NC + cid
        cnt = base_cnt + jnp.where(wid < n_extra, 1, 0)
        pltpu.sync_copy(x4_hbm, xtab)

        @pl.loop(0, base_cnt + 1)
        def _(i):
            @pl.when(i < cnt)
            def _():
                base = (wid + _NW * i) * _K
                pltpu.sync_copy(row_hbm.at[pl.ds(base, _K)], rowi)
                pltpu.sync_copy(col_hbm.at[pl.ds(base, _K)], coli)
                cpa = pltpu.async_copy(pab_hbm.at[rowi], bufr, sema)
                cpb = pltpu.async_copy(pab_hbm.at[coli], bufc, semb)
                cpa.wait()
                cpb.wait()

                @pl.loop(0, _K, step=16)
                def _(g):
                    rv = rowi[pl.ds(g, 16)]
                    cv = coli[pl.ds(g, 16)]
                    d0 = (plsc.load_gather(xtab, [rv, _full16(0)])
                          - plsc.load_gather(xtab, [cv, _full16(0)]))
                    d1 = (plsc.load_gather(xtab, [rv, _full16(1)])
                          - plsc.load_gather(xtab, [cv, _full16(1)]))
                    d2 = (plsc.load_gather(xtab, [rv, _full16(2)])
                          - plsc.load_gather(xtab, [cv, _full16(2)]))
                    rad = d0 * d0 + d1 * d1 + d2 * d2
                    rows = lax.iota(I32, 16) + g
                    plsc.store_scatter(bufr, [rows, _full16(64)], rad)
                    plsc.store_scatter(bufr, [rows, _full16(65)], d0)
                    plsc.store_scatter(bufr, [rows, _full16(66)], d1)
                    plsc.store_scatter(bufr, [rows, _full16(67)], d2)

                @pl.loop(0, _K)
                def _(e):
                    for c in (0, 16, 32, 48):
                        bufr[e, pl.ds(c, 16)] = (
                            bufr[e, pl.ds(c, 16)]
                            + bufc[e, pl.ds(c + 64, 16)])

                pltpu.sync_copy(bufr, gpre_hbm.at[pl.ds(base, _K)])

    return gather_k


def _make_scatter(E, N):
    n_chunks = E // _K
    base_cnt = n_chunks // _NW
    n_extra = n_chunks - base_cnt * _NW
    rows_per_tile = N // _NS
    zrows = rows_per_tile // 5
    mesh = plsc.VectorSubcoreMesh(core_axis_name="c", subcore_axis_name="s")

    @functools.partial(
        pl.kernel,
        mesh=mesh,
        out_type=jax.ShapeDtypeStruct((_NC, N, 128), F32),
        scratch_types=[
            pltpu.VMEM_SHARED((N, 128), F32),
            pltpu.VMEM((1, _K), I32),
            pltpu.VMEM((_K, 128), F32),
            pltpu.VMEM((zrows, 128), F32),
        ],
        compiler_params=_sc_params(tc_tiling=False),
    )
    def scatter_k(row_hbm, oe_hbm, part_hbm, acc, rowi, ebuf, zbuf):
        cid = lax.axis_index("c")
        sid = lax.axis_index("s")
        wid = sid * _NC + cid
        cnt = base_cnt + jnp.where(wid < n_extra, 1, 0)

        z16 = jnp.zeros((16,), F32)

        @pl.loop(0, zrows)
        def _(r):
            @pl.loop(0, 128, step=16)
            def _(c):
                zbuf[r, pl.ds(c, 16)] = z16

        @pl.loop(0, 5)
        def _(j):
            off = sid * rows_per_tile + j * zrows
            pltpu.sync_copy(zbuf, acc.at[pl.ds(off, zrows)])

        plsc.subcore_barrier()

        @pl.loop(0, base_cnt + 1)
        def _(i):
            @pl.when(i < cnt)
            def _():
                base = (wid + _NW * i) * _K
                pltpu.sync_copy(row_hbm.at[pl.ds(base, _K)], rowi.at[0])
                pltpu.sync_copy(oe_hbm.at[pl.ds(base, _K)], ebuf)
                pltpu.sync_copy(ebuf, acc.at[rowi.at[0]], add=True)

        plsc.subcore_barrier()

        off = sid * rows_per_tile
        pltpu.sync_copy(acc.at[pl.ds(off, rows_per_tile)],
                        part_hbm.at[cid, pl.ds(off, rows_per_tile)])

    return scatter_k


def _init_tc(his, emb_W, emb_b, W1a, W1b):
    N, D = his.shape
    BN = 1000

    def body(his_r, ew_r, eb_r, wa_r, wb_r, h_r, pab_r):
        h = jnp.dot(his_r[...], ew_r[...], precision=_PREC) + eb_r[...]
        h_r[...] = h
        pab_r[...] = jnp.concatenate(
            [jnp.dot(h, wa_r[...], precision=_PREC),
             jnp.dot(h, wb_r[...], precision=_PREC)], axis=1)

    return pl.pallas_call(
        body,
        grid=(N // BN,),
        in_specs=[
            pl.BlockSpec((BN, D), lambda i: (i, 0)),
            pl.BlockSpec((D, 64), lambda i: (0, 0)),
            pl.BlockSpec((1, 64), lambda i: (0, 0)),
            pl.BlockSpec((64, 64), lambda i: (0, 0)),
            pl.BlockSpec((64, 64), lambda i: (0, 0)),
        ],
        out_specs=[
            pl.BlockSpec((BN, 64), lambda i: (i, 0)),
            pl.BlockSpec((BN, 128), lambda i: (i, 0)),
        ],
        out_shape=[
            jax.ShapeDtypeStruct((N, 64), F32),
            jax.ShapeDtypeStruct((N, 128), F32),
        ],
    )(his, emb_W, emb_b, W1a, W1b)


def _eaproj_tc(edge_attr, W1d, b1):
    E = edge_attr.shape[0]
    Eh = E // 2
    BE = 640

    def body(lo_r, hi_r, w_r, b_r, o_r):
        o_r[...] = jnp.concatenate(
            [jnp.dot(lo_r[...], w_r[...], precision=_PREC) + b_r[...],
             jnp.dot(hi_r[...], w_r[...], precision=_PREC) + b_r[...]],
            axis=1)

    nb = Eh // BE
    return pl.pallas_call(
        body,
        grid=(nb,),
        in_specs=[
            pl.BlockSpec((BE, 16), lambda i: (i, 0)),
            pl.BlockSpec((BE, 16), lambda i: (i + nb, 0)),
            pl.BlockSpec((16, 64), lambda i: (0, 0)),
            pl.BlockSpec((1, 64), lambda i: (0, 0)),
        ],
        out_specs=pl.BlockSpec((BE, 128), lambda i: (i, 0)),
        out_shape=jax.ShapeDtypeStruct((Eh, 128), F32),
    )(edge_attr, edge_attr, W1d, b1)


def _edge_tc(gpre, eap, w1c, W2, b2, cW1, cb1, cW2r):
    E = gpre.shape[0]
    Eh = E // 2
    BE = 640
    nb = Eh // BE

    def half(gp, ea, w1c_v, w2_v, b2_v, cw1_v, cb1_v, cw2_v):
        pre = gp[:, 0:64] + ea + gp[:, 64:65] * w1c_v
        m = _silu(jnp.dot(_silu(pre), w2_v, precision=_PREC) + b2_v)
        t = _silu(jnp.dot(m, cw1_v, precision=_PREC) + cb1_v)
        phi = jnp.sum(t * cw2_v, axis=1, keepdims=True)
        trans = gp[:, 65:68] * phi
        return jnp.concatenate(
            [m, trans, jnp.ones((BE, 1), F32), jnp.zeros((BE, 60), F32)],
            axis=1)

    def body(glo_r, ghi_r, eap_r, w1c_r, w2_r, b2_r, cw1_r, cb1_r, cw2_r,
             oe_r):
        eapv = eap_r[...]
        oe_r[0] = half(glo_r[...], eapv[:, 0:64], w1c_r[...], w2_r[...],
                       b2_r[...], cw1_r[...], cb1_r[...], cw2_r[...])
        oe_r[1] = half(ghi_r[...], eapv[:, 64:128], w1c_r[...], w2_r[...],
                       b2_r[...], cw1_r[...], cb1_r[...], cw2_r[...])

    oe2 = pl.pallas_call(
        body,
        grid=(nb,),
        in_specs=[
            pl.BlockSpec((BE, 128), lambda i: (i, 0)),
            pl.BlockSpec((BE, 128), lambda i: (i + nb, 0)),
            pl.BlockSpec((BE, 128), lambda i: (i, 0)),
            pl.BlockSpec((1, 64), lambda i: (0, 0)),
            pl.BlockSpec((64, 64), lambda i: (0, 0)),
            pl.BlockSpec((1, 64), lambda i: (0, 0)),
            pl.BlockSpec((64, 64), lambda i: (0, 0)),
            pl.BlockSpec((1, 64), lambda i: (0, 0)),
            pl.BlockSpec((1, 64), lambda i: (0, 0)),
        ],
        out_specs=pl.BlockSpec((2, BE, 128), lambda i: (0, i, 0)),
        out_shape=jax.ShapeDtypeStruct((2, Eh, 128), F32),
    )(gpre, gpre, eap, w1c, W2, b2, cW1, cb1, cW2r)
    return oe2.reshape(E, 128)


def _node_tc(part, h, x4, v4, vW1, vb1, vW2r, vb2, nW1a, nW1b, nb1,
             nW2, nb2, W1a, W1b):
    N = h.shape[0]
    BN = 1000

    def body(p_r, h_r, x4_r, v4_r, vw1_r, vb1_r, vw2_r, vb2_r,
             nwa_r, nwb_r, nb1_r, nw2_r, nb2_r, wa_r, wb_r,
             hn_r, xn_r, vn_r, pab_r):
        h = h_r[...]
        pv = p_r[...]
        ptot = pv[0] + pv[1]
        aggm = ptot[:, 0:64]
        agg4 = ptot[:, 64:68] / jnp.maximum(ptot[:, 67:68], 1.0)
        vs1 = _silu(jnp.dot(h, vw1_r[...], precision=_PREC) + vb1_r[...])
        vscale = jnp.sum(vs1 * vw2_r[...], axis=1, keepdims=True) + vb2_r[...]
        v4n = vscale * v4_r[...] + agg4
        x4n = x4_r[...] + v4n
        hn1 = _silu(jnp.dot(h, nwa_r[...], precision=_PREC)
                    + jnp.dot(aggm, nwb_r[...], precision=_PREC) + nb1_r[...])
        hnn = jnp.dot(hn1, nw2_r[...], precision=_PREC) + nb2_r[...]
        h2 = 2.0 * h + hnn
        hn_r[...] = h2
        xn_r[...] = x4n
        vn_r[...] = v4n
        pab_r[...] = jnp.concatenate(
            [jnp.dot(h2, wa_r[...], precision=_PREC),
             jnp.dot(h2, wb_r[...], precision=_PREC)], axis=1)

    w64 = pl.BlockSpec((64, 64), lambda i: (0, 0))
    b64 = pl.BlockSpec((1, 64), lambda i: (0, 0))
    return pl.pallas_call(
        body,
        grid=(N // BN,),
        in_specs=[
            pl.BlockSpec((2, BN, 128), lambda i: (0, i, 0)),
            pl.BlockSpec((BN, 64), lambda i: (i, 0)),
            pl.BlockSpec((BN, 4), lambda i: (i, 0)),
            pl.BlockSpec((BN, 4), lambda i: (i, 0)),
            w64, b64, b64,
            pl.BlockSpec((1, 1), lambda i: (0, 0)),
            w64, w64, b64, w64, b64, w64, w64,
        ],
        out_specs=[
            pl.BlockSpec((BN, 64), lambda i: (i, 0)),
            pl.BlockSpec((BN, 4), lambda i: (i, 0)),
            pl.BlockSpec((BN, 4), lambda i: (i, 0)),
            pl.BlockSpec((BN, 128), lambda i: (i, 0)),
        ],
        out_shape=[
            jax.ShapeDtypeStruct((N, 64), F32),
            jax.ShapeDtypeStruct((N, 4), F32),
            jax.ShapeDtypeStruct((N, 4), F32),
            jax.ShapeDtypeStruct((N, 128), F32),
        ],
    )(part, h, x4, v4, vW1, vb1, vW2r, vb2, nW1a, nW1b, nb1, nW2, nb2,
      W1a, W1b)


def kernel(his, x, v, edges, edge_attr, params):
    p = params
    N = his.shape[0]
    E = edge_attr.shape[0]
    row = edges[0]
    col = edges[1]

    W1 = p['e_W1']
    W1a, W1b = W1[0:64], W1[64:128]
    w1c = W1[128:129]
    W1d = W1[129:145]
    nW1 = p['n_W1']
    nW1a, nW1b = nW1[0:64], nW1[64:128]
    emb_b = p['emb_b'].reshape(1, 64)
    e_b1 = p['e_b1'].reshape(1, 64)
    e_b2 = p['e_b2'].reshape(1, 64)
    c_b1 = p['c_b1'].reshape(1, 64)
    cW2r = p['c_W2'].reshape(1, 64)
    v_b1 = p['v_b1'].reshape(1, 64)
    vW2r = p['v_W2'].reshape(1, 64)
    v_b2 = p['v_b2'].reshape(1, 1)
    n_b1 = p['n_b1'].reshape(1, 64)
    n_b2 = p['n_b2'].reshape(1, 64)

    x4 = jnp.pad(x, ((0, 0), (0, 1)))
    v4 = jnp.pad(v, ((0, 0), (0, 1)))

    gather_k = _make_gather(E, N)
    scatter_k = _make_scatter(E, N)

    h, pab = _init_tc(his, p['emb_W'], emb_b, W1a, W1b)
    eap = _eaproj_tc(edge_attr, W1d, e_b1)

    for _ in range(3):
        gpre = gather_k(pab, x4, row, col)
        oe = _edge_tc(gpre, eap, w1c, p['e_W2'], e_b2,
                      p['c_W1'], c_b1, cW2r)
        part = scatter_k(row, oe)
        h, x4, v4, pab = _node_tc(part, h, x4, v4, p['v_W1'], v_b1,
                                  vW2r, v_b2, nW1a, nW1b, n_b1,
                                  p['n_W2'], n_b2, W1a, W1b)

    return (x4[:, :3], h, v4[:, :3])
